# Initial kernel scaffold; baseline (speedup 1.0000x reference)
import jax
import jax.numpy as jnp
from jax.experimental import pallas as pl


def kernel(x, W1, W2, W3, W4, W5, g1, g2, g3, g4, g5, b1, b2, b3, b4, b5):
    raise NotImplementedError("write your pallas kernel here")


# speedup vs baseline: 6.8254x; 6.8254x over previous
import functools

import jax
import jax.numpy as jnp
from jax import lax
from jax.experimental import pallas as pl
from jax.experimental.pallas import tpu as pltpu
from jax.experimental.pallas import tpu_sc as plsc

B = 8
N = 2048
KNN = 20
KPAD = 32
EPS = 1e-5
ALPHA = 0.2
BLK = 256
RBLK = 1024
NEG = -3.0e38
F32 = jnp.float32
_PREC = lax.Precision.HIGHEST


def _knn_proj_body(x_ref, wat_ref, wbt_ref, idx_ref, zt_ref, wt_ref, s1_ref,
                   sq_ref, s_ref, *, cn_layout, cin):
    b = pl.program_id(0)
    i = pl.program_id(1)
    def _score_dot(a, bm, dims):
        return lax.dot_general(a, bm, (dims, ((), ())),
                               preferred_element_type=F32)

    if cn_layout:
        xf = x_ref[0]
        xb = x_ref[0, :, pl.ds(i * BLK, BLK)]
        g = _score_dot(xb, xf, ((0,), (0,)))
        xsq = xf * xf
        xx = jnp.sum(xsq, axis=0, keepdims=True)
        xx_col = lax.dot_general(
            xb * xb, jnp.ones((cin, 1), F32), (((0,), (0,)), ((), ())),
            preferred_element_type=F32, precision=_PREC)
        zt = lax.dot_general(xb, wat_ref[...], (((0,), (0,)), ((), ())),
                             preferred_element_type=F32)
        wt = lax.dot_general(xb, wbt_ref[...], (((0,), (0,)), ((), ())),
                             preferred_element_type=F32)
    else:
        xf = x_ref[0]
        xb = x_ref[0, pl.ds(i * BLK, BLK), :]
        g = _score_dot(xb, xf, ((1,), (1,)))
        xx = lax.dot_general(jnp.ones((1, cin), F32), xf * xf,
                             (((1,), (1,)), ((), ())),
                             preferred_element_type=F32, precision=_PREC)
        xx_col = jnp.sum(xb * xb, axis=1, keepdims=True)
        zt = lax.dot_general(xb, wat_ref[...], (((1,), (0,)), ((), ())),
                             preferred_element_type=F32)
        wt = lax.dot_general(xb, wbt_ref[...], (((1,), (0,)), ((), ())),
                             preferred_element_type=F32)
    inner = -2.0 * g
    s_ref[...] = (-xx_col - inner) - xx

    col_ids = lax.broadcasted_iota(jnp.int32, (BLK, N), 1)
    lane_ids = lax.broadcasted_iota(jnp.int32, (BLK, KPAD), 1)
    base = b * N

    def step(t, acc):
        m = jnp.max(s_ref[...], axis=1, keepdims=True)
        j = jnp.min(jnp.where(s_ref[...] >= m, col_ids, N), axis=1,
                    keepdims=True)
        s_ref[...] = jnp.where(col_ids == j, NEG, s_ref[...])
        return jnp.where(lane_ids == t, j + base, acc)

    idx_ref[0] = lax.fori_loop(0, KNN, step,
                               jnp.zeros((BLK, KPAD), jnp.int32))[:, :KNN]
    msk = jnp.where(s_ref[...] <= 0.5 * NEG, 1.0, 0.0)
    zt_ref[0] = zt
    wt_ref[0] = wt
    if cn_layout:
        ztf = lax.dot_general(xf, wat_ref[...], (((0,), (0,)), ((), ())),
                              preferred_element_type=F32)
    else:
        ztf = lax.dot_general(xf, wat_ref[...], (((1,), (0,)), ((), ())),
                              preferred_element_type=F32)
    co = wt.shape[1]
    s1_ref[0] = lax.dot_general(msk, ztf, (((1,), (0,)), ((), ())),
                                preferred_element_type=F32,
                                precision=_PREC)[:, :co]
    sq_ref[0] = lax.dot_general(msk, ztf * ztf, (((1,), (0,)), ((), ())),
                                preferred_element_type=F32,
                                precision=_PREC)[:, :co]


def _knn_proj(x, wat, wbt, cn_layout):
    cin, co = wbt.shape
    co_pad = wat.shape[1]
    grid = (B, N // BLK)
    xspec = pl.BlockSpec((1,) + x.shape[1:], lambda b, i: (b, 0, 0))
    return pl.pallas_call(
        functools.partial(_knn_proj_body, cn_layout=cn_layout, cin=cin),
        grid=grid,
        in_specs=[xspec,
                  pl.BlockSpec((cin, co_pad), lambda b, i: (0, 0)),
                  pl.BlockSpec((cin, co), lambda b, i: (0, 0))],
        out_specs=[
            pl.BlockSpec((1, BLK, KNN), lambda b, i: (b, i, 0)),
            pl.BlockSpec((1, BLK, co_pad), lambda b, i: (b, i, 0)),
            pl.BlockSpec((1, BLK, co), lambda b, i: (b, i, 0)),
            pl.BlockSpec((1, BLK, co), lambda b, i: (b, i, 0)),
            pl.BlockSpec((1, BLK, co), lambda b, i: (b, i, 0)),
        ],
        out_shape=[
            jax.ShapeDtypeStruct((B, N, KNN), jnp.int32),
            jax.ShapeDtypeStruct((B, N, co_pad), F32),
            jax.ShapeDtypeStruct((B, N, co), F32),
            jax.ShapeDtypeStruct((B, N, co), F32),
            jax.ShapeDtypeStruct((B, N, co), F32),
        ],
        scratch_shapes=[pltpu.VMEM((BLK, N), F32)],
    )(x, wat, wbt)


def _gather_reduce(zt_flat, idx_flat, co):
    bn = B * N
    co_pad = zt_flat.shape[1]
    pch = 2 if co > 128 else 4
    idxc = pch * KNN
    info = plsc.get_sparse_core_info()
    nw = info.num_cores * info.num_subcores
    ppw = bn // nw
    nch = ppw // pch
    mesh = plsc.VectorSubcoreMesh(core_axis_name="c", subcore_axis_name="s")

    @functools.partial(
        pl.kernel, mesh=mesh,
        out_type=jax.ShapeDtypeStruct((bn * co,), F32),
        scratch_types=[
            pltpu.VMEM((ppw * KNN,), jnp.int32),
            pltpu.VMEM((idxc, co_pad), F32),
            pltpu.VMEM((idxc, co_pad), F32),
            pltpu.VMEM((pch * co,), F32),
            pltpu.VMEM((pch * co,), F32),
            pltpu.SemaphoreType.DMA,
            pltpu.SemaphoreType.DMA,
            pltpu.SemaphoreType.DMA,
            pltpu.SemaphoreType.DMA,
        ])
    def k2(zt_hbm, idx_hbm, out_hbm, idx_all, rows0, rows1, outv0, outv1,
           gs0, gs1, os0, os1):
        wid = lax.axis_index("s") * info.num_cores + lax.axis_index("c")
        pbase = wid * ppw
        pltpu.sync_copy(idx_hbm.at[pl.ds(pbase * KNN, ppw * KNN)], idx_all)
        rows = (rows0, rows1)
        outv = (outv0, outv1)
        gs = (gs0, gs1)
        os = (os0, os1)

        def issue(g, buf):
            pltpu.async_copy(
                zt_hbm.at[idx_all.at[pl.ds(g * idxc, idxc)]],
                rows[buf], gs[buf])

        issue(0, 0)

        def half(g, par):
            @pl.when(g + 1 < nch)
            def _():
                issue(g + 1, 1 - par)

            pltpu.make_async_copy(
                zt_hbm.at[idx_all.at[pl.ds(0, idxc)]], rows[par],
                gs[par]).wait()

            @pl.when(g >= 2)
            def _():
                pltpu.make_async_copy(
                    outv[par], out_hbm.at[pl.ds(0, pch * co)],
                    os[par]).wait()

            for p in range(pch):
                for cc in range(co // 16):
                    sl = pl.ds(cc * 16, 16)
                    mx = rows[par][p * KNN, sl]
                    for j in range(1, KNN):
                        mx = jnp.maximum(mx, rows[par][p * KNN + j, sl])
                    outv[par][pl.ds(p * co + cc * 16, 16)] = mx
            pltpu.async_copy(
                outv[par],
                out_hbm.at[pl.ds((pbase + g * pch) * co, pch * co)],
                os[par])

        def outer(t, carry):
            half(2 * t, 0)
            half(2 * t + 1, 1)
            return carry

        lax.fori_loop(0, nch // 2, outer, 0)
        for par in range(2):
            pltpu.make_async_copy(
                outv[par], out_hbm.at[pl.ds(0, pch * co)], os[par]).wait()

    return k2(zt_flat, idx_flat)


def _stats_sums(s1_flat, sq_flat, wt_flat, co):
    bn = B * N

    def body(s1_ref, sq_ref, wt_ref, out_ref):
        pid = pl.program_id(0)
        s1 = s1_ref[...]
        sq = sq_ref[...]
        w = wt_ref[...]
        blk = jnp.concatenate([
            jnp.sum(s1, axis=0, keepdims=True),
            jnp.sum(sq, axis=0, keepdims=True),
            jnp.sum(w * s1, axis=0, keepdims=True),
            jnp.sum(w, axis=0, keepdims=True),
            jnp.sum(w * w, axis=0, keepdims=True),
            jnp.zeros((3, co), F32),
        ], axis=0)

        @pl.when(pid == 0)
        def _():
            out_ref[...] = blk

        @pl.when(pid != 0)
        def _():
            out_ref[...] = out_ref[...] + blk

    return pl.pallas_call(
        body, grid=(bn // RBLK,),
        in_specs=[pl.BlockSpec((RBLK, co), lambda i: (i, 0)),
                  pl.BlockSpec((RBLK, co), lambda i: (i, 0)),
                  pl.BlockSpec((RBLK, co), lambda i: (i, 0))],
        out_specs=pl.BlockSpec((8, co), lambda i: (0, 0)),
        out_shape=jax.ShapeDtypeStruct((8, co), F32),
    )(s1_flat, sq_flat, wt_flat)


def _stats_final(sums, gamma, beta, co):
    count = float(B * N * KNN)

    def body(s_ref, g_ref, b_ref, out_ref):
        s = s_ref[...]
        inv = 1.0 / count
        mean = (s[0:1] + KNN * s[3:4]) * inv
        ey2 = (s[1:2] + 2.0 * s[2:3] + KNN * s[4:5]) * inv
        var = ey2 - mean * mean
        sd = jnp.sqrt(var + EPS)
        out_ref[...] = jnp.concatenate(
            [mean, sd, g_ref[...], b_ref[...], jnp.zeros((4, co), F32)],
            axis=0)

    return pl.pallas_call(
        body,
        in_specs=[pl.BlockSpec((8, co), lambda: (0, 0)),
                  pl.BlockSpec((1, co), lambda: (0, 0)),
                  pl.BlockSpec((1, co), lambda: (0, 0))],
        out_specs=pl.BlockSpec((8, co), lambda: (0, 0)),
        out_shape=jax.ShapeDtypeStruct((8, co), F32),
    )(sums, gamma.reshape(1, co), beta.reshape(1, co))


def _apply_act(m_flat, wt_flat, ss, co):
    bn = B * N

    def body(m_ref, wt_ref, ss_ref, out_ref):
        xn = (m_ref[...] + wt_ref[...] - ss_ref[0:1, :]) / ss_ref[1:2, :]
        v = xn * ss_ref[2:3, :] + ss_ref[3:4, :]
        out_ref[...] = jnp.maximum(v, ALPHA * v)

    return pl.pallas_call(
        body, grid=(bn // RBLK,),
        in_specs=[pl.BlockSpec((RBLK, co), lambda i: (i, 0)),
                  pl.BlockSpec((RBLK, co), lambda i: (i, 0)),
                  pl.BlockSpec((8, co), lambda i: (0, 0))],
        out_specs=pl.BlockSpec((RBLK, co), lambda i: (i, 0)),
        out_shape=jax.ShapeDtypeStruct((bn, co), F32),
    )(m_flat, wt_flat, ss)


def _edge_layer(x, wat, wbt, gamma, beta, cn_layout):
    cin, co = wbt.shape
    co_pad = max(co, 128)
    wat_pad = jnp.concatenate(
        [wat, jnp.zeros((cin, co_pad - co), F32)], axis=1) if co_pad > co else wat
    idx, zt, wt, s1, sq = _knn_proj(x, wat_pad, wbt, cn_layout)
    m = _gather_reduce(zt.reshape(B * N, co_pad),
                       idx.reshape(B * N * KNN), co)
    m_flat = m.reshape(B * N, co)
    wt_flat = wt.reshape(B * N, co)
    sums = _stats_sums(s1.reshape(B * N, co), sq.reshape(B * N, co),
                       wt_flat, co)
    ss = _stats_final(sums, gamma, beta, co)
    xt = _apply_act(m_flat, wt_flat, ss, co)
    return xt.reshape(B, N, co)


def _final_matmul(x1f, x2f, x3f, x4f, w5t):
    bn = B * N

    def body(a_ref, b_ref, c_ref, d_ref, w_ref, y_ref, s_ref):
        pid = pl.program_id(0)
        cat = jnp.concatenate(
            [a_ref[...], b_ref[...], c_ref[...], d_ref[...]], axis=1)
        y = lax.dot_general(cat, w_ref[...], (((1,), (0,)), ((), ())),
                            preferred_element_type=F32)
        y_ref[...] = y
        blk = jnp.concatenate([
            jnp.sum(y, axis=0, keepdims=True),
            jnp.sum(y * y, axis=0, keepdims=True),
            jnp.zeros((6, 512), F32),
        ], axis=0)

        @pl.when(pid == 0)
        def _():
            s_ref[...] = blk

        @pl.when(pid != 0)
        def _():
            s_ref[...] = s_ref[...] + blk

    return pl.pallas_call(
        body, grid=(bn // RBLK,),
        in_specs=[pl.BlockSpec((RBLK, 64), lambda i: (i, 0)),
                  pl.BlockSpec((RBLK, 64), lambda i: (i, 0)),
                  pl.BlockSpec((RBLK, 128), lambda i: (i, 0)),
                  pl.BlockSpec((RBLK, 256), lambda i: (i, 0)),
                  pl.BlockSpec((512, 512), lambda i: (0, 0))],
        out_specs=[pl.BlockSpec((RBLK, 512), lambda i: (i, 0)),
                   pl.BlockSpec((8, 512), lambda i: (0, 0))],
        out_shape=[jax.ShapeDtypeStruct((bn, 512), F32),
                   jax.ShapeDtypeStruct((8, 512), F32)],
    )(x1f, x2f, x3f, x4f, w5t)


def _final_stats(sums, gamma, beta):
    count = float(B * N)

    def body(s_ref, g_ref, b_ref, out_ref):
        inv = 1.0 / count
        mean = s_ref[0:1] * inv
        var = s_ref[1:2] * inv - mean * mean
        sd = jnp.sqrt(var + EPS)
        out_ref[...] = jnp.concatenate(
            [mean, sd, g_ref[...], b_ref[...], jnp.zeros((4, 512), F32)],
            axis=0)

    return pl.pallas_call(
        body,
        in_specs=[pl.BlockSpec((8, 512), lambda: (0, 0)),
                  pl.BlockSpec((1, 512), lambda: (0, 0)),
                  pl.BlockSpec((1, 512), lambda: (0, 0))],
        out_specs=pl.BlockSpec((8, 512), lambda: (0, 0)),
        out_shape=jax.ShapeDtypeStruct((8, 512), F32),
    )(sums, gamma.reshape(1, 512), beta.reshape(1, 512))


def _final_apply(y, ss):
    def body(y_ref, ss_ref, out_ref):
        xn = (y_ref[0] - ss_ref[0:1, :]) / ss_ref[1:2, :]
        v = xn * ss_ref[2:3, :] + ss_ref[3:4, :]
        v = jnp.maximum(v, ALPHA * v)
        out_ref[0] = jnp.swapaxes(v, 0, 1)

    return pl.pallas_call(
        body, grid=(B, N // BLK),
        in_specs=[pl.BlockSpec((1, BLK, 512), lambda b, i: (b, i, 0)),
                  pl.BlockSpec((8, 512), lambda b, i: (0, 0))],
        out_specs=pl.BlockSpec((1, 512, BLK), lambda b, i: (b, 0, i)),
        out_shape=jax.ShapeDtypeStruct((B, 512, N), F32),
    )(y.reshape(B, N, 512), ss)


def kernel(x, W1, W2, W3, W4, W5, g1, g2, g3, g4, g5, b1, b2, b3, b4, b5):
    x1t = _edge_layer(x, W1[:, :3].T, W1[:, 3:].T, g1, b1, cn_layout=True)
    x2t = _edge_layer(x1t, W2[:, :64].T, W2[:, 64:].T, g2, b2, cn_layout=False)
    x3t = _edge_layer(x2t, W3[:, :64].T, W3[:, 64:].T, g3, b3, cn_layout=False)
    x4t = _edge_layer(x3t, W4[:, :128].T, W4[:, 128:].T, g4, b4,
                      cn_layout=False)
    y, ysums = _final_matmul(x1t.reshape(B * N, 64), x2t.reshape(B * N, 64),
                             x3t.reshape(B * N, 128), x4t.reshape(B * N, 256),
                             W5.T)
    ss = _final_stats(ysums, g5, b5)
    out = _final_apply(y, ss)
    return (x, out)


# speedup vs baseline: 8.3037x; 1.2166x over previous
import functools

import jax
import jax.numpy as jnp
from jax import lax
from jax.experimental import pallas as pl
from jax.experimental.pallas import tpu as pltpu
from jax.experimental.pallas import tpu_sc as plsc

B = 8
N = 2048
KNN = 20
KPAD = 32
EPS = 1e-5
ALPHA = 0.2
BLK = 256
RBLK = 1024
NEG = -3.0e38
F32 = jnp.float32
_PREC = lax.Precision.HIGHEST


def _knn_proj_body(x_ref, wat_ref, wbt_ref, idx_ref, zt_ref, wt_ref, s1_ref,
                   sq_ref, s_ref, *, cn_layout, cin):
    b = pl.program_id(0)
    i = pl.program_id(1)
    def _score_dot(a, bm, dims):
        return lax.dot_general(a, bm, (dims, ((), ())),
                               preferred_element_type=F32)

    if cn_layout:
        xf = x_ref[0]
        xb = x_ref[0, :, pl.ds(i * BLK, BLK)]
        g = _score_dot(xb, xf, ((0,), (0,)))
        xsq = xf * xf
        xx = jnp.sum(xsq, axis=0, keepdims=True)
        xx_col = lax.dot_general(
            xb * xb, jnp.ones((cin, 1), F32), (((0,), (0,)), ((), ())),
            preferred_element_type=F32, precision=_PREC)
        zt = lax.dot_general(xb, wat_ref[...], (((0,), (0,)), ((), ())),
                             preferred_element_type=F32)
        wt = lax.dot_general(xb, wbt_ref[...], (((0,), (0,)), ((), ())),
                             preferred_element_type=F32)
    else:
        xf = x_ref[0]
        xb = x_ref[0, pl.ds(i * BLK, BLK), :]
        g = _score_dot(xb, xf, ((1,), (1,)))
        xx = lax.dot_general(jnp.ones((1, cin), F32), xf * xf,
                             (((1,), (1,)), ((), ())),
                             preferred_element_type=F32, precision=_PREC)
        xx_col = jnp.sum(xb * xb, axis=1, keepdims=True)
        zt = lax.dot_general(xb, wat_ref[...], (((1,), (0,)), ((), ())),
                             preferred_element_type=F32)
        wt = lax.dot_general(xb, wbt_ref[...], (((1,), (0,)), ((), ())),
                             preferred_element_type=F32)
    inner = -2.0 * g
    s_ref[...] = (-xx_col - inner) - xx

    col_ids = lax.broadcasted_iota(jnp.int32, (BLK, N), 1)
    lane_ids = lax.broadcasted_iota(jnp.int32, (BLK, KPAD), 1)
    base = b * N

    def step(t, carry):
        jprev, acc = carry
        masked = jnp.where(col_ids == jprev, NEG, s_ref[...])
        s_ref[...] = masked
        m = jnp.max(masked, axis=1, keepdims=True)
        j = jnp.min(jnp.where(masked >= m, col_ids, N), axis=1,
                    keepdims=True)
        return j, jnp.where(lane_ids == t, j + base, acc)

    jlast, acc = lax.fori_loop(
        0, KNN, step,
        (jnp.full((BLK, 1), N, jnp.int32), jnp.zeros((BLK, KPAD), jnp.int32)))
    idx_ref[0] = acc[:, :KNN]
    msk = jnp.where((s_ref[...] <= 0.5 * NEG) | (col_ids == jlast), 1.0, 0.0)
    zt_ref[0] = zt
    wt_ref[0] = wt
    if cn_layout:
        ztf = lax.dot_general(xf, wat_ref[...], (((0,), (0,)), ((), ())),
                              preferred_element_type=F32)
    else:
        ztf = lax.dot_general(xf, wat_ref[...], (((1,), (0,)), ((), ())),
                              preferred_element_type=F32)
    co = wt.shape[1]

    def _mask_dot(mat):
        hi = mat.astype(jnp.bfloat16)
        lo = (mat - hi.astype(F32)).astype(jnp.bfloat16)
        mb = msk.astype(jnp.bfloat16)
        dims = (((1,), (0,)), ((), ()))
        return (lax.dot_general(mb, hi, dims, preferred_element_type=F32)
                + lax.dot_general(mb, lo, dims, preferred_element_type=F32))

    s1_ref[0] = _mask_dot(ztf)[:, :co]
    sq_ref[0] = _mask_dot(ztf * ztf)[:, :co]


def _knn_proj(x, wat, wbt, cn_layout):
    cin, co = wbt.shape
    co_pad = wat.shape[1]
    nb = x.shape[0]
    grid = (nb, N // BLK)
    xspec = pl.BlockSpec((1,) + x.shape[1:], lambda b, i: (b, 0, 0))
    return pl.pallas_call(
        functools.partial(_knn_proj_body, cn_layout=cn_layout, cin=cin),
        grid=grid,
        in_specs=[xspec,
                  pl.BlockSpec((cin, co_pad), lambda b, i: (0, 0)),
                  pl.BlockSpec((cin, co), lambda b, i: (0, 0))],
        out_specs=[
            pl.BlockSpec((1, BLK, KNN), lambda b, i: (b, i, 0)),
            pl.BlockSpec((1, BLK, co_pad), lambda b, i: (b, i, 0)),
            pl.BlockSpec((1, BLK, co), lambda b, i: (b, i, 0)),
            pl.BlockSpec((1, BLK, co), lambda b, i: (b, i, 0)),
            pl.BlockSpec((1, BLK, co), lambda b, i: (b, i, 0)),
        ],
        out_shape=[
            jax.ShapeDtypeStruct((nb, N, KNN), jnp.int32),
            jax.ShapeDtypeStruct((nb, N, co_pad), F32),
            jax.ShapeDtypeStruct((nb, N, co), F32),
            jax.ShapeDtypeStruct((nb, N, co), F32),
            jax.ShapeDtypeStruct((nb, N, co), F32),
        ],
        scratch_shapes=[pltpu.VMEM((BLK, N), F32)],
    )(x, wat, wbt)


def _gather_reduce(zt_flat, idx_flat, co):
    bn = zt_flat.shape[0]
    co_pad = zt_flat.shape[1]
    pch = 2 if co > 128 else 4
    idxc = pch * KNN
    info = plsc.get_sparse_core_info()
    nw = info.num_cores * info.num_subcores
    ppw = bn // nw
    nch = ppw // pch
    mesh = plsc.VectorSubcoreMesh(core_axis_name="c", subcore_axis_name="s")

    @functools.partial(
        pl.kernel, mesh=mesh,
        out_type=jax.ShapeDtypeStruct((bn * co,), F32),
        scratch_types=[
            pltpu.VMEM((ppw * KNN,), jnp.int32),
            pltpu.VMEM((idxc, co_pad), F32),
            pltpu.VMEM((idxc, co_pad), F32),
            pltpu.VMEM((pch * co,), F32),
            pltpu.VMEM((pch * co,), F32),
            pltpu.SemaphoreType.DMA,
            pltpu.SemaphoreType.DMA,
            pltpu.SemaphoreType.DMA,
            pltpu.SemaphoreType.DMA,
        ])
    def k2(zt_hbm, idx_hbm, out_hbm, idx_all, rows0, rows1, outv0, outv1,
           gs0, gs1, os0, os1):
        wid = lax.axis_index("s") * info.num_cores + lax.axis_index("c")
        pbase = wid * ppw
        pltpu.sync_copy(idx_hbm.at[pl.ds(pbase * KNN, ppw * KNN)], idx_all)
        rows = (rows0, rows1)
        outv = (outv0, outv1)
        gs = (gs0, gs1)
        os = (os0, os1)

        def issue(g, buf):
            pltpu.async_copy(
                zt_hbm.at[idx_all.at[pl.ds(g * idxc, idxc)]],
                rows[buf], gs[buf])

        issue(0, 0)

        def half(g, par):
            @pl.when(g + 1 < nch)
            def _():
                issue(g + 1, 1 - par)

            pltpu.make_async_copy(
                zt_hbm.at[idx_all.at[pl.ds(0, idxc)]], rows[par],
                gs[par]).wait()

            @pl.when(g >= 2)
            def _():
                pltpu.make_async_copy(
                    outv[par], out_hbm.at[pl.ds(0, pch * co)],
                    os[par]).wait()

            for p in range(pch):
                for cc in range(co // 16):
                    sl = pl.ds(cc * 16, 16)
                    mx = rows[par][p * KNN, sl]
                    for j in range(1, KNN):
                        mx = jnp.maximum(mx, rows[par][p * KNN + j, sl])
                    outv[par][pl.ds(p * co + cc * 16, 16)] = mx
            pltpu.async_copy(
                outv[par],
                out_hbm.at[pl.ds((pbase + g * pch) * co, pch * co)],
                os[par])

        def outer(t, carry):
            half(2 * t, 0)
            half(2 * t + 1, 1)
            return carry

        lax.fori_loop(0, nch // 2, outer, 0)
        for par in range(2):
            pltpu.make_async_copy(
                outv[par], out_hbm.at[pl.ds(0, pch * co)], os[par]).wait()

    return k2(zt_flat, idx_flat)


def _stats_sums(s1_flat, sq_flat, wt_flat, co):
    bn = B * N

    def body(s1_ref, sq_ref, wt_ref, out_ref):
        pid = pl.program_id(0)
        s1 = s1_ref[...]
        sq = sq_ref[...]
        w = wt_ref[...]
        blk = jnp.concatenate([
            jnp.sum(s1, axis=0, keepdims=True),
            jnp.sum(sq, axis=0, keepdims=True),
            jnp.sum(w * s1, axis=0, keepdims=True),
            jnp.sum(w, axis=0, keepdims=True),
            jnp.sum(w * w, axis=0, keepdims=True),
            jnp.zeros((3, co), F32),
        ], axis=0)

        @pl.when(pid == 0)
        def _():
            out_ref[...] = blk

        @pl.when(pid != 0)
        def _():
            out_ref[...] = out_ref[...] + blk

    return pl.pallas_call(
        body, grid=(bn // RBLK,),
        in_specs=[pl.BlockSpec((RBLK, co), lambda i: (i, 0)),
                  pl.BlockSpec((RBLK, co), lambda i: (i, 0)),
                  pl.BlockSpec((RBLK, co), lambda i: (i, 0))],
        out_specs=pl.BlockSpec((8, co), lambda i: (0, 0)),
        out_shape=jax.ShapeDtypeStruct((8, co), F32),
    )(s1_flat, sq_flat, wt_flat)


def _stats_final(sums, gamma, beta, co):
    count = float(B * N * KNN)

    def body(s_ref, g_ref, b_ref, out_ref):
        s = s_ref[...]
        inv = 1.0 / count
        mean = (s[0:1] + KNN * s[3:4]) * inv
        ey2 = (s[1:2] + 2.0 * s[2:3] + KNN * s[4:5]) * inv
        var = ey2 - mean * mean
        sd = jnp.sqrt(var + EPS)
        out_ref[...] = jnp.concatenate(
            [mean, sd, g_ref[...], b_ref[...], jnp.zeros((4, co), F32)],
            axis=0)

    return pl.pallas_call(
        body,
        in_specs=[pl.BlockSpec((8, co), lambda: (0, 0)),
                  pl.BlockSpec((1, co), lambda: (0, 0)),
                  pl.BlockSpec((1, co), lambda: (0, 0))],
        out_specs=pl.BlockSpec((8, co), lambda: (0, 0)),
        out_shape=jax.ShapeDtypeStruct((8, co), F32),
    )(sums, gamma.reshape(1, co), beta.reshape(1, co))


def _apply_act(m_flat, wt_flat, ss, co):
    bn = B * N

    def body(m_ref, wt_ref, ss_ref, out_ref):
        xn = (m_ref[...] + wt_ref[...] - ss_ref[0:1, :]) / ss_ref[1:2, :]
        v = xn * ss_ref[2:3, :] + ss_ref[3:4, :]
        out_ref[...] = jnp.maximum(v, ALPHA * v)

    return pl.pallas_call(
        body, grid=(bn // RBLK,),
        in_specs=[pl.BlockSpec((RBLK, co), lambda i: (i, 0)),
                  pl.BlockSpec((RBLK, co), lambda i: (i, 0)),
                  pl.BlockSpec((8, co), lambda i: (0, 0))],
        out_specs=pl.BlockSpec((RBLK, co), lambda i: (i, 0)),
        out_shape=jax.ShapeDtypeStruct((bn, co), F32),
    )(m_flat, wt_flat, ss)


def _edge_layer(x, wat, wbt, gamma, beta, cn_layout, groups=4):
    cin, co = wbt.shape
    co_pad = max(co, 128)
    wat_pad = jnp.concatenate(
        [wat, jnp.zeros((cin, co_pad - co), F32)], axis=1) if co_pad > co else wat
    nb = B // groups
    ms, s1s, sqs, wts = [], [], [], []
    for gi in range(groups):
        xg = x[gi * nb:(gi + 1) * nb]
        idx, zt, wt, s1, sq = _knn_proj(xg, wat_pad, wbt, cn_layout)
        m = _gather_reduce(zt.reshape(nb * N, co_pad),
                           idx.reshape(nb * N * KNN), co)
        ms.append(m.reshape(nb * N, co))
        s1s.append(s1.reshape(nb * N, co))
        sqs.append(sq.reshape(nb * N, co))
        wts.append(wt.reshape(nb * N, co))
    m_flat = jnp.concatenate(ms, axis=0)
    wt_flat = jnp.concatenate(wts, axis=0)
    sums = _stats_sums(jnp.concatenate(s1s, axis=0),
                       jnp.concatenate(sqs, axis=0), wt_flat, co)
    ss = _stats_final(sums, gamma, beta, co)
    xt = _apply_act(m_flat, wt_flat, ss, co)
    return xt.reshape(B, N, co)


def _final_matmul(x1f, x2f, x3f, x4f, w5t):
    bn = B * N

    def body(a_ref, b_ref, c_ref, d_ref, w_ref, y_ref, s_ref):
        pid = pl.program_id(0)
        cat = jnp.concatenate(
            [a_ref[...], b_ref[...], c_ref[...], d_ref[...]], axis=1)
        y = lax.dot_general(cat, w_ref[...], (((1,), (0,)), ((), ())),
                            preferred_element_type=F32)
        y_ref[...] = y
        blk = jnp.concatenate([
            jnp.sum(y, axis=0, keepdims=True),
            jnp.sum(y * y, axis=0, keepdims=True),
            jnp.zeros((6, 512), F32),
        ], axis=0)

        @pl.when(pid == 0)
        def _():
            s_ref[...] = blk

        @pl.when(pid != 0)
        def _():
            s_ref[...] = s_ref[...] + blk

    return pl.pallas_call(
        body, grid=(bn // RBLK,),
        in_specs=[pl.BlockSpec((RBLK, 64), lambda i: (i, 0)),
                  pl.BlockSpec((RBLK, 64), lambda i: (i, 0)),
                  pl.BlockSpec((RBLK, 128), lambda i: (i, 0)),
                  pl.BlockSpec((RBLK, 256), lambda i: (i, 0)),
                  pl.BlockSpec((512, 512), lambda i: (0, 0))],
        out_specs=[pl.BlockSpec((RBLK, 512), lambda i: (i, 0)),
                   pl.BlockSpec((8, 512), lambda i: (0, 0))],
        out_shape=[jax.ShapeDtypeStruct((bn, 512), F32),
                   jax.ShapeDtypeStruct((8, 512), F32)],
    )(x1f, x2f, x3f, x4f, w5t)


def _final_stats(sums, gamma, beta):
    count = float(B * N)

    def body(s_ref, g_ref, b_ref, out_ref):
        inv = 1.0 / count
        mean = s_ref[0:1] * inv
        var = s_ref[1:2] * inv - mean * mean
        sd = jnp.sqrt(var + EPS)
        out_ref[...] = jnp.concatenate(
            [mean, sd, g_ref[...], b_ref[...], jnp.zeros((4, 512), F32)],
            axis=0)

    return pl.pallas_call(
        body,
        in_specs=[pl.BlockSpec((8, 512), lambda: (0, 0)),
                  pl.BlockSpec((1, 512), lambda: (0, 0)),
                  pl.BlockSpec((1, 512), lambda: (0, 0))],
        out_specs=pl.BlockSpec((8, 512), lambda: (0, 0)),
        out_shape=jax.ShapeDtypeStruct((8, 512), F32),
    )(sums, gamma.reshape(1, 512), beta.reshape(1, 512))


def _final_apply(y, ss):
    def body(y_ref, ss_ref, out_ref):
        xn = (y_ref[0] - ss_ref[0:1, :]) / ss_ref[1:2, :]
        v = xn * ss_ref[2:3, :] + ss_ref[3:4, :]
        v = jnp.maximum(v, ALPHA * v)
        out_ref[0] = jnp.swapaxes(v, 0, 1)

    return pl.pallas_call(
        body, grid=(B, N // BLK),
        in_specs=[pl.BlockSpec((1, BLK, 512), lambda b, i: (b, i, 0)),
                  pl.BlockSpec((8, 512), lambda b, i: (0, 0))],
        out_specs=pl.BlockSpec((1, 512, BLK), lambda b, i: (b, 0, i)),
        out_shape=jax.ShapeDtypeStruct((B, 512, N), F32),
    )(y.reshape(B, N, 512), ss)


def kernel(x, W1, W2, W3, W4, W5, g1, g2, g3, g4, g5, b1, b2, b3, b4, b5):
    x1t = _edge_layer(x, W1[:, :3].T, W1[:, 3:].T, g1, b1, cn_layout=True)
    x2t = _edge_layer(x1t, W2[:, :64].T, W2[:, 64:].T, g2, b2, cn_layout=False)
    x3t = _edge_layer(x2t, W3[:, :64].T, W3[:, 64:].T, g3, b3, cn_layout=False)
    x4t = _edge_layer(x3t, W4[:, :128].T, W4[:, 128:].T, g4, b4,
                      cn_layout=False)
    y, ysums = _final_matmul(x1t.reshape(B * N, 64), x2t.reshape(B * N, 64),
                             x3t.reshape(B * N, 128), x4t.reshape(B * N, 256),
                             W5.T)
    ss = _final_stats(ysums, g5, b5)
    out = _final_apply(y, ss)
    return (x, out)
